# Initial kernel scaffold; baseline (speedup 1.0000x reference)
import jax
import jax.numpy as jnp
from jax.experimental import pallas as pl


def kernel(image_resnet, params, A, ref_vertices):
    raise NotImplementedError("write your pallas kernel here")


# speedup vs baseline: 2.6221x; 2.6221x over previous
import jax
import jax.numpy as jnp
from jax import lax
from jax.experimental import pallas as pl


def _full_spec(a):
    nd = a.ndim
    return pl.BlockSpec(a.shape, lambda b, _nd=nd: (0,) * _nd)


def kernel(image_resnet, params, A, ref_vertices):
    f32 = jnp.float32
    B, D = image_resnet.shape
    n = ref_vertices.shape[0]
    npad = -(-n // 128) * 128

    A_pad = jnp.pad(A, ((0, npad - n), (0, npad - n)))
    refv = jnp.pad(ref_vertices, ((0, npad - n), (0, 0)))
    img3 = image_resnet.reshape(B, 1, D)

    args = [img3, refv, A_pad]
    specs = [pl.BlockSpec((1, 1, D), lambda b: (b, 0, 0)), _full_spec(refv),
             _full_spec(A_pad)]

    def add(a):
        args.append(a)
        specs.append(_full_spec(a))

    W0 = params["lin0"]["W"]
    add(W0[:, :3].T)
    add(W0[:, 3:].T)
    add(params["lin0"]["b"].reshape(1, -1))

    blocks = list(params["gc_blocks"]) + list(params["shape_blocks"])
    has_skip = []
    for p in blocks:
        add(p["pre_norm"]["gamma"].reshape(1, -1))
        add(p["pre_norm"]["beta"].reshape(1, -1))
        add(p["lin1"]["W"].T)
        add(p["lin1"]["b"].reshape(1, -1))
        add(p["norm1"]["gamma"].reshape(1, -1))
        add(p["norm1"]["beta"].reshape(1, -1))
        add(p["conv"]["W"])
        add(p["conv"]["b"].reshape(1, -1))
        add(p["norm2"]["gamma"].reshape(1, -1))
        add(p["norm2"]["beta"].reshape(1, -1))
        add(p["lin2"]["W"].T)
        add(p["lin2"]["b"].reshape(1, -1))
        hs = "skip" in p
        has_skip.append(hs)
        if hs:
            add(p["skip"]["W"].T)
            add(p["skip"]["b"].reshape(1, -1))

    add(params["final_gn"]["gamma"].reshape(1, -1))
    add(params["final_gn"]["beta"].reshape(1, -1))
    add(params["final_lin"]["W"])
    add(params["final_lin"]["b"].reshape(-1, 1))

    nf = float(n)

    def body(*refs):
        img_ref, ref_ref, A_ref = refs[:3]
        out_ref = refs[-1]
        it = iter(refs[3:-1])

        def nxt():
            return next(it)[...]

        mask = (lax.broadcasted_iota(jnp.int32, (npad, 1), 0) < n).astype(f32)

        def dot(a, b):
            return jnp.dot(a, b, preferred_element_type=f32)

        def gn_relu(x, g, bb):
            C = x.shape[1]
            ii = lax.broadcasted_iota(jnp.int32, (C, C), 0) // 8
            jj = lax.broadcasted_iota(jnp.int32, (C, C), 1) // 8
            M = (ii == jj).astype(f32)
            s = jnp.sum(x, axis=0, keepdims=True)
            s2 = jnp.sum(x * x, axis=0, keepdims=True)
            cnt = 8.0 * nf
            mean = dot(s, M) / cnt
            var = dot(s2, M) / cnt - mean * mean
            sc = lax.rsqrt(var + 1e-5) * g
            sh = bb - mean * sc
            return jnp.maximum(x * sc + sh, 0.0) * mask

        w3t, wimgt, b0 = nxt(), nxt(), nxt()
        x = (dot(ref_ref[...], w3t) + dot(img_ref[0], wimgt) + b0) * mask

        for hs in has_skip:
            gp, bp = nxt(), nxt()
            w1, b1 = nxt(), nxt()
            g1, be1 = nxt(), nxt()
            wc, bc = nxt(), nxt()
            g2, be2 = nxt(), nxt()
            w2, b2 = nxt(), nxt()
            y = gn_relu(x, gp, bp)
            y = (dot(y, w1) + b1) * mask
            y = gn_relu(y, g1, be1)
            sup = dot(y, wc)
            z = (dot(A_ref[...], sup) + bc) * mask
            z = gn_relu(z, g2, be2)
            y2 = dot(z, w2) + b2
            if hs:
                ws, bs = nxt(), nxt()
                xs = dot(x, ws) + bs
            else:
                xs = x
            x = (xs + y2) * mask

        gf, bf = nxt(), nxt()
        wf, bfin = nxt(), nxt()
        y = gn_relu(x, gf, bf)
        outT = lax.dot_general(wf, y, (((1,), (1,)), ((), ())),
                               preferred_element_type=f32)
        out_ref[0] = (outT + bfin)[:, :n]

    out = pl.pallas_call(
        body,
        grid=(B,),
        in_specs=specs,
        out_specs=pl.BlockSpec((1, 3, n), lambda b: (b, 0, 0)),
        out_shape=jax.ShapeDtypeStruct((B, 3, n), f32),
    )(*args)
    return out


# speedup vs baseline: 2.6257x; 1.0014x over previous
import jax
import jax.numpy as jnp
from jax import lax
from jax.experimental import pallas as pl
from jax.experimental.pallas import tpu as pltpu


def _full_spec(a):
    nd = a.ndim
    return pl.BlockSpec(a.shape, lambda b, _nd=nd: (0,) * _nd)


def kernel(image_resnet, params, A, ref_vertices):
    f32 = jnp.float32
    B, D = image_resnet.shape
    n = ref_vertices.shape[0]
    npad = -(-n // 128) * 128

    A_pad = jnp.pad(A, ((0, npad - n), (0, npad - n)))
    refv = jnp.pad(ref_vertices, ((0, npad - n), (0, 0)))
    img3 = image_resnet.reshape(B, 1, D)

    args = [img3, refv, A_pad]
    specs = [pl.BlockSpec((1, 1, D), lambda b: (b, 0, 0)), _full_spec(refv),
             _full_spec(A_pad)]

    def add(a):
        args.append(a)
        specs.append(_full_spec(a))

    W0 = params["lin0"]["W"]
    add(W0[:, :3].T)
    add(W0[:, 3:].T)
    add(params["lin0"]["b"].reshape(1, -1))

    blocks = list(params["gc_blocks"]) + list(params["shape_blocks"])
    has_skip = []
    for p in blocks:
        add(p["pre_norm"]["gamma"].reshape(1, -1))
        add(p["pre_norm"]["beta"].reshape(1, -1))
        add(p["lin1"]["W"].T)
        add(p["lin1"]["b"].reshape(1, -1))
        add(p["norm1"]["gamma"].reshape(1, -1))
        add(p["norm1"]["beta"].reshape(1, -1))
        add(p["conv"]["W"])
        add(p["conv"]["b"].reshape(1, -1))
        add(p["norm2"]["gamma"].reshape(1, -1))
        add(p["norm2"]["beta"].reshape(1, -1))
        add(p["lin2"]["W"].T)
        add(p["lin2"]["b"].reshape(1, -1))
        hs = "skip" in p
        has_skip.append(hs)
        if hs:
            add(p["skip"]["W"].T)
            add(p["skip"]["b"].reshape(1, -1))

    add(params["final_gn"]["gamma"].reshape(1, -1))
    add(params["final_gn"]["beta"].reshape(1, -1))
    add(params["final_lin"]["W"])
    add(params["final_lin"]["b"].reshape(-1, 1))

    nf = float(n)

    def body(*refs):
        img_ref, ref_ref, A_ref = refs[:3]
        out_ref = refs[-1]
        it = iter(refs[3:-1])

        def nxt():
            return next(it)[...]

        mask = (lax.broadcasted_iota(jnp.int32, (npad, 1), 0) < n).astype(f32)

        def dot(a, b):
            return jnp.dot(a, b, preferred_element_type=f32)

        def gn_relu(x, g, bb):
            C = x.shape[1]
            ii = lax.broadcasted_iota(jnp.int32, (C, C), 0) // 8
            jj = lax.broadcasted_iota(jnp.int32, (C, C), 1) // 8
            M = (ii == jj).astype(f32)
            s = jnp.sum(x, axis=0, keepdims=True)
            s2 = jnp.sum(x * x, axis=0, keepdims=True)
            cnt = 8.0 * nf
            mean = dot(s, M) / cnt
            var = dot(s2, M) / cnt - mean * mean
            sc = lax.rsqrt(var + 1e-5) * g
            sh = bb - mean * sc
            return jnp.maximum(x * sc + sh, 0.0) * mask

        w3t, wimgt, b0 = nxt(), nxt(), nxt()
        x = (dot(ref_ref[...], w3t) + dot(img_ref[0], wimgt) + b0) * mask

        for hs in has_skip:
            gp, bp = nxt(), nxt()
            w1, b1 = nxt(), nxt()
            g1, be1 = nxt(), nxt()
            wc, bc = nxt(), nxt()
            g2, be2 = nxt(), nxt()
            w2, b2 = nxt(), nxt()
            y = gn_relu(x, gp, bp)
            y = (dot(y, w1) + b1) * mask
            y = gn_relu(y, g1, be1)
            sup = dot(y, wc)
            z = (dot(A_ref[...], sup) + bc) * mask
            z = gn_relu(z, g2, be2)
            y2 = dot(z, w2) + b2
            if hs:
                ws, bs = nxt(), nxt()
                xs = dot(x, ws) + bs
            else:
                xs = x
            x = (xs + y2) * mask

        gf, bf = nxt(), nxt()
        wf, bfin = nxt(), nxt()
        y = gn_relu(x, gf, bf)
        outT = lax.dot_general(wf, y, (((1,), (1,)), ((), ())),
                               preferred_element_type=f32)
        out_ref[0] = (outT + bfin)[:, :n]

    out = pl.pallas_call(
        body,
        grid=(B,),
        in_specs=specs,
        out_specs=pl.BlockSpec((1, 3, n), lambda b: (b, 0, 0)),
        out_shape=jax.ShapeDtypeStruct((B, 3, n), f32),
        compiler_params=pltpu.CompilerParams(
            dimension_semantics=("parallel",)),
    )(*args)
    return out
